# Initial kernel scaffold; baseline (speedup 1.0000x reference)
import jax
import jax.numpy as jnp
from jax.experimental import pallas as pl


def kernel(indices, emb_table, W1, b1, W2, b2):
    raise NotImplementedError("write your pallas kernel here")


# speedup vs baseline: 1.2624x; 1.2624x over previous
import functools

import jax
import jax.numpy as jnp
from jax import lax
from jax.experimental import pallas as pl
from jax.experimental.pallas import tpu as pltpu
from jax.experimental.pallas import tpu_sc as plsc

D = 128
SEQ = 200
B = 4096
HID = 300
NL = 2

NC, NS = 2, 16
NW = NC * NS
BPW = B // NW
CHUNKS = (104, 96)


def _sc_avg(indices, emb_table):
    mesh = plsc.VectorSubcoreMesh(core_axis_name="c", subcore_axis_name="s")

    @functools.partial(
        pl.kernel,
        mesh=mesh,
        compiler_params=pltpu.CompilerParams(use_tc_tiling_on_sc=False),
        out_type=jax.ShapeDtypeStruct((B, D), jnp.float32),
        scratch_types=[
            pltpu.VMEM((BPW, SEQ), jnp.int32),
            pltpu.VMEM((SEQ, D), jnp.float32),
            pltpu.VMEM((BPW, D), jnp.float32),
            pltpu.SemaphoreType.DMA,
        ],
    )
    def k(idx_hbm, tbl_hbm, out_hbm, idx_v, rows_v, out_v, sem):
        wid = lax.axis_index("s") * NC + lax.axis_index("c")
        base = wid * BPW
        pltpu.sync_copy(idx_hbm.at[pl.ds(base, BPW)], idx_v)

        def body_b(b, carry):
            off = 0
            copies = []
            for ch in CHUNKS:
                copies.append(
                    pltpu.async_copy(
                        tbl_hbm.at[idx_v.at[b, pl.ds(off, ch)]],
                        rows_v.at[pl.ds(off, ch)],
                        sem,
                    )
                )
                off += ch
            for c in copies:
                c.wait()

            def body_s(s, acc):
                return tuple(
                    acc[d] + rows_v[s, pl.ds(d * 16, 16)] for d in range(8)
                )

            acc = lax.fori_loop(
                0, SEQ, body_s,
                tuple(jnp.zeros((16,), jnp.float32) for _ in range(8)),
            )
            scale = jnp.float32(1.0 / SEQ)
            for d in range(8):
                out_v[b, pl.ds(d * 16, 16)] = acc[d] * scale
            return carry

        lax.fori_loop(0, BPW, body_b, 0)
        pltpu.sync_copy(out_v, out_hbm.at[pl.ds(base, BPW)])

    return k(indices, emb_table)


def _tc_mlp(avg, W1, b1, W2, b2):
    BT = 512

    def body(avg_ref, w1_ref, b1_ref, w2_ref, b2_ref, out_ref):
        h = jnp.dot(avg_ref[...], w1_ref[...],
                    preferred_element_type=jnp.float32) + b1_ref[...]
        h = 0.5 * h * (1.0 + lax.erf(h * jnp.float32(0.7071067811865476)))
        out_ref[...] = jnp.dot(h, w2_ref[...],
                               preferred_element_type=jnp.float32) + b2_ref[...]

    return pl.pallas_call(
        body,
        grid=(B // BT,),
        in_specs=[
            pl.BlockSpec((BT, D), lambda i: (i, 0)),
            pl.BlockSpec((D, HID), lambda i: (0, 0)),
            pl.BlockSpec((1, HID), lambda i: (0, 0)),
            pl.BlockSpec((HID, NL), lambda i: (0, 0)),
            pl.BlockSpec((1, NL), lambda i: (0, 0)),
        ],
        out_specs=pl.BlockSpec((BT, NL), lambda i: (i, 0)),
        out_shape=jax.ShapeDtypeStruct((B, NL), jnp.float32),
    )(avg, W1, b1, W2, b2)


def kernel(indices, emb_table, W1, b1, W2, b2):
    avg = _sc_avg(indices.astype(jnp.int32), emb_table)
    return _tc_mlp(avg, W1, b1.reshape(1, HID), W2, b2.reshape(1, NL))


# speedup vs baseline: 2.1722x; 1.7207x over previous
import functools

import jax
import jax.numpy as jnp
from jax import lax
from jax.experimental import pallas as pl
from jax.experimental.pallas import tpu as pltpu
from jax.experimental.pallas import tpu_sc as plsc

D = 128
SEQ = 200
B = 4096
HID = 300
NL = 2

NC, NS = 2, 16
NW = NC * NS
BPW = B // NW
CHUNKS = (104, 96)


def _sc_avg(indices, emb_table):
    mesh = plsc.VectorSubcoreMesh(core_axis_name="c", subcore_axis_name="s")

    @functools.partial(
        pl.kernel,
        mesh=mesh,
        compiler_params=pltpu.CompilerParams(use_tc_tiling_on_sc=False),
        out_type=jax.ShapeDtypeStruct((B, D), jnp.float32),
        scratch_types=[
            pltpu.VMEM((BPW, SEQ), jnp.int32),
            pltpu.VMEM((2, SEQ, D), jnp.float32),
            pltpu.VMEM((BPW, D), jnp.float32),
            pltpu.SemaphoreType.DMA,
            pltpu.SemaphoreType.DMA,
        ],
    )
    def k(idx_hbm, tbl_hbm, out_hbm, idx_v, rows_v, out_v, sem0, sem1):
        wid = lax.axis_index("s") * NC + lax.axis_index("c")
        base = wid * BPW
        sems = (sem0, sem1)
        pltpu.sync_copy(idx_hbm.at[pl.ds(base, BPW)], idx_v)

        def gather_row(b, j, sem):
            off = 0
            for ch in CHUNKS:
                pltpu.async_copy(
                    tbl_hbm.at[idx_v.at[b, pl.ds(off, ch)]],
                    rows_v.at[j, pl.ds(off, ch)],
                    sem,
                )
                off += ch

        def wait_row(j, sem):
            pltpu.make_async_copy(
                tbl_hbm.at[pl.ds(0, SEQ)], rows_v.at[j], sem
            ).wait()

        gather_row(0, 0, sems[0])
        gather_row(1, 1, sems[1])

        def body_pair(i, carry):
            for j in range(2):
                b = 2 * i + j
                wait_row(j, sems[j])

                def body_s(si, acc):
                    accs = list(acc)
                    for u in range(8):
                        s = si * 8 + u
                        for d in range(8):
                            accs[d] = accs[d] + rows_v[j, s, pl.ds(d * 16, 16)]
                    return tuple(accs)

                acc = lax.fori_loop(
                    0, SEQ // 8, body_s,
                    tuple(jnp.zeros((16,), jnp.float32) for _ in range(8)),
                )
                scale = jnp.float32(1.0 / SEQ)
                for d in range(8):
                    out_v[b, pl.ds(d * 16, 16)] = acc[d] * scale

                @pl.when(b + 2 < BPW)
                def _():
                    gather_row(b + 2, j, sems[j])
            return carry

        lax.fori_loop(0, BPW // 2, body_pair, 0)
        pltpu.sync_copy(out_v, out_hbm.at[pl.ds(base, BPW)])

    return k(indices, emb_table)


def _tc_mlp(avg, W1, b1, W2, b2):
    BT = 512

    def body(avg_ref, w1_ref, b1_ref, w2_ref, b2_ref, out_ref):
        h = jnp.dot(avg_ref[...], w1_ref[...],
                    preferred_element_type=jnp.float32) + b1_ref[...]
        h = 0.5 * h * (1.0 + lax.erf(h * jnp.float32(0.7071067811865476)))
        out_ref[...] = jnp.dot(h, w2_ref[...],
                               preferred_element_type=jnp.float32) + b2_ref[...]

    return pl.pallas_call(
        body,
        grid=(B // BT,),
        in_specs=[
            pl.BlockSpec((BT, D), lambda i: (i, 0)),
            pl.BlockSpec((D, HID), lambda i: (0, 0)),
            pl.BlockSpec((1, HID), lambda i: (0, 0)),
            pl.BlockSpec((HID, NL), lambda i: (0, 0)),
            pl.BlockSpec((1, NL), lambda i: (0, 0)),
        ],
        out_specs=pl.BlockSpec((BT, NL), lambda i: (i, 0)),
        out_shape=jax.ShapeDtypeStruct((B, NL), jnp.float32),
    )(avg, W1, b1, W2, b2)


def kernel(indices, emb_table, W1, b1, W2, b2):
    avg = _sc_avg(indices.astype(jnp.int32), emb_table)
    return _tc_mlp(avg, W1, b1.reshape(1, HID), W2, b2.reshape(1, NL))


# speedup vs baseline: 2.5656x; 1.1811x over previous
import functools

import jax
import jax.numpy as jnp
from jax import lax
from jax.experimental import pallas as pl
from jax.experimental.pallas import tpu as pltpu
from jax.experimental.pallas import tpu_sc as plsc

D = 128
SEQ = 200
B = 4096
HID = 300
NL = 2

NC, NS = 2, 16
NW = NC * NS
BPW = B // NW
CHUNKS = (104, 96)


def _sc_avg(indices, emb_table):
    mesh = plsc.VectorSubcoreMesh(core_axis_name="c", subcore_axis_name="s")

    @functools.partial(
        pl.kernel,
        mesh=mesh,
        compiler_params=pltpu.CompilerParams(use_tc_tiling_on_sc=False),
        out_type=jax.ShapeDtypeStruct((B, D), jnp.float32),
        scratch_types=[
            pltpu.VMEM((8, SEQ), jnp.int32),
            pltpu.VMEM((4, SEQ, D), jnp.float32),
            pltpu.VMEM((BPW, D), jnp.float32),
            [pltpu.SemaphoreType.DMA] * 4,
            [pltpu.SemaphoreType.DMA] * 8,
        ],
    )
    def k(idx_hbm, tbl_hbm, out_hbm, idx_v, rows_v, out_v, sem_rows, sem_idx):
        wid = lax.axis_index("s") * NC + lax.axis_index("c")
        base = wid * BPW

        def fetch_idx(b, jj):
            pltpu.async_copy(idx_hbm.at[base + b], idx_v.at[jj], sem_idx[jj])

        def wait_idx(jj):
            pltpu.make_async_copy(
                idx_hbm.at[0], idx_v.at[jj], sem_idx[jj]
            ).wait()

        def gather_row(b, j, jj):
            off = 0
            for ch in CHUNKS:
                pltpu.async_copy(
                    tbl_hbm.at[idx_v.at[jj, pl.ds(off, ch)]],
                    rows_v.at[j, pl.ds(off, ch)],
                    sem_rows[j],
                )
                off += ch

        def wait_row(j):
            pltpu.make_async_copy(
                tbl_hbm.at[pl.ds(0, SEQ)], rows_v.at[j], sem_rows[j]
            ).wait()

        for jj in range(8):
            fetch_idx(jj, jj)
        for j in range(4):
            wait_idx(j)
            gather_row(j, j, j)

        def body_oct(i, carry):
            for j in range(8):
                b = 8 * i + j
                wait_row(j % 4)

                def body_s(si, acc):
                    accs = list(acc)
                    for u in range(8):
                        s = si * 8 + u
                        for d in range(8):
                            accs[d] = accs[d] + rows_v[j % 4, s, pl.ds(d * 16, 16)]
                    return tuple(accs)

                acc = lax.fori_loop(
                    0, SEQ // 8, body_s,
                    tuple(jnp.zeros((16,), jnp.float32) for _ in range(8)),
                )
                scale = jnp.float32(1.0 / SEQ)
                for d in range(8):
                    out_v[b, pl.ds(d * 16, 16)] = acc[d] * scale

                @pl.when(b + 4 < BPW)
                def _():
                    wait_idx((j + 4) % 8)
                    gather_row(b + 4, j % 4, (j + 4) % 8)

                @pl.when(b + 8 < BPW)
                def _():
                    fetch_idx(b + 8, j)
            return carry

        lax.fori_loop(0, BPW // 8, body_oct, 0)
        pltpu.sync_copy(out_v, out_hbm.at[pl.ds(base, BPW)])

    return k(indices, emb_table)


def _tc_mlp(avg, W1, b1, W2, b2):
    BT = 512

    def body(avg_ref, w1_ref, b1_ref, w2_ref, b2_ref, out_ref):
        h = jnp.dot(avg_ref[...], w1_ref[...],
                    preferred_element_type=jnp.float32) + b1_ref[...]
        h = 0.5 * h * (1.0 + lax.erf(h * jnp.float32(0.7071067811865476)))
        out_ref[...] = jnp.dot(h, w2_ref[...],
                               preferred_element_type=jnp.float32) + b2_ref[...]

    return pl.pallas_call(
        body,
        grid=(B // BT,),
        in_specs=[
            pl.BlockSpec((BT, D), lambda i: (i, 0)),
            pl.BlockSpec((D, HID), lambda i: (0, 0)),
            pl.BlockSpec((1, HID), lambda i: (0, 0)),
            pl.BlockSpec((HID, NL), lambda i: (0, 0)),
            pl.BlockSpec((1, NL), lambda i: (0, 0)),
        ],
        out_specs=pl.BlockSpec((BT, NL), lambda i: (i, 0)),
        out_shape=jax.ShapeDtypeStruct((B, NL), jnp.float32),
    )(avg, W1, b1, W2, b2)


def kernel(indices, emb_table, W1, b1, W2, b2):
    avg = _sc_avg(indices.astype(jnp.int32), emb_table)
    return _tc_mlp(avg, W1, b1.reshape(1, HID), W2, b2.reshape(1, NL))


# speedup vs baseline: 2.6023x; 1.0143x over previous
import functools

import jax
import jax.numpy as jnp
from jax import lax
from jax.experimental import pallas as pl
from jax.experimental.pallas import tpu as pltpu
from jax.experimental.pallas import tpu_sc as plsc

D = 128
SEQ = 200
B = 4096
HID = 300
NL = 2

NC, NS = 2, 16
NW = NC * NS
BPW = B // NW
CHUNKS = (104, 96)


def _sc_avg(indices, emb_table):
    mesh = plsc.VectorSubcoreMesh(core_axis_name="c", subcore_axis_name="s")

    @functools.partial(
        pl.kernel,
        mesh=mesh,
        compiler_params=pltpu.CompilerParams(use_tc_tiling_on_sc=False),
        out_type=jax.ShapeDtypeStruct((B, D), jnp.float32),
        scratch_types=[
            pltpu.VMEM((8, SEQ), jnp.int32),
            pltpu.VMEM((4, SEQ, D), jnp.float32),
            pltpu.VMEM((BPW, D), jnp.float32),
            [pltpu.SemaphoreType.DMA] * 4,
            [pltpu.SemaphoreType.DMA] * 8,
        ],
    )
    def k(idx_hbm, tbl_hbm, out_hbm, idx_v, rows_v, out_v, sem_rows, sem_idx):
        wid = lax.axis_index("s") * NC + lax.axis_index("c")
        base = wid * BPW

        def fetch_idx(b, jj):
            pltpu.async_copy(idx_hbm.at[base + b], idx_v.at[jj], sem_idx[jj])

        def wait_idx(jj):
            pltpu.make_async_copy(
                idx_hbm.at[0], idx_v.at[jj], sem_idx[jj]
            ).wait()

        def gather_row(b, j, jj):
            off = 0
            for ch in CHUNKS:
                pltpu.async_copy(
                    tbl_hbm.at[idx_v.at[jj, pl.ds(off, ch)]],
                    rows_v.at[j, pl.ds(off, ch)],
                    sem_rows[j],
                )
                off += ch

        def wait_row(j):
            pltpu.make_async_copy(
                tbl_hbm.at[pl.ds(0, sum(CHUNKS))],
                rows_v.at[j, pl.ds(0, sum(CHUNKS))],
                sem_rows[j],
            ).wait()

        for jj in range(8):
            fetch_idx(jj, jj)
        for j in range(4):
            wait_idx(j)
            gather_row(j, j, j)

        def body_oct(i, carry):
            for j in range(8):
                b = 8 * i + j
                wait_row(j % 4)

                def body_s(si, acc):
                    accs = list(acc)
                    for u in range(8):
                        s = si * 8 + u
                        for d in range(8):
                            accs[d] = accs[d] + rows_v[j % 4, s, pl.ds(d * 16, 16)]
                    return tuple(accs)

                acc = lax.fori_loop(
                    0, SEQ // 8, body_s,
                    tuple(jnp.zeros((16,), jnp.float32) for _ in range(8)),
                )
                scale = jnp.float32(1.0 / SEQ)
                for d in range(8):
                    out_v[b, pl.ds(d * 16, 16)] = acc[d] * scale

                @pl.when(b + 4 < BPW)
                def _():
                    wait_idx((j + 4) % 8)
                    gather_row(b + 4, j % 4, (j + 4) % 8)

                @pl.when(b + 8 < BPW)
                def _():
                    fetch_idx(b + 8, j)
            return carry

        lax.fori_loop(0, BPW // 8, body_oct, 0)
        pltpu.sync_copy(out_v, out_hbm.at[pl.ds(base, BPW)])

    return k(indices, emb_table)


def _tc_mlp(avg, W1, b1, W2, b2):
    BT = 1024

    def body(avg_ref, w1_ref, b1_ref, w2_ref, b2_ref, out_ref):
        h = jnp.dot(avg_ref[...], w1_ref[...],
                    preferred_element_type=jnp.float32) + b1_ref[...]
        h = 0.5 * h * (1.0 + lax.erf(h * jnp.float32(0.7071067811865476)))
        out_ref[...] = jnp.dot(h, w2_ref[...],
                               preferred_element_type=jnp.float32) + b2_ref[...]

    return pl.pallas_call(
        body,
        grid=(B // BT,),
        in_specs=[
            pl.BlockSpec((BT, D), lambda i: (i, 0)),
            pl.BlockSpec((D, HID), lambda i: (0, 0)),
            pl.BlockSpec((1, HID), lambda i: (0, 0)),
            pl.BlockSpec((HID, NL), lambda i: (0, 0)),
            pl.BlockSpec((1, NL), lambda i: (0, 0)),
        ],
        out_specs=pl.BlockSpec((BT, NL), lambda i: (i, 0)),
        out_shape=jax.ShapeDtypeStruct((B, NL), jnp.float32),
    )(avg, W1, b1, W2, b2)


def kernel(indices, emb_table, W1, b1, W2, b2):
    avg = _sc_avg(indices.astype(jnp.int32), emb_table)
    return _tc_mlp(avg, W1, b1.reshape(1, HID), W2, b2.reshape(1, NL))
